# Initial kernel scaffold; baseline (speedup 1.0000x reference)
import jax
import jax.numpy as jnp
from jax.experimental import pallas as pl


def kernel(x, W1, b1, W2, b2, W3, b3):
    raise NotImplementedError("write your pallas kernel here")


# speedup vs baseline: 93.4401x; 93.4401x over previous
import functools

import jax
import jax.numpy as jnp
import numpy as np
from jax.experimental import pallas as pl
from jax.experimental.pallas import tpu as pltpu

_H = 128
_N = _H * _H
_PAD = 640
_RB = 1024
_NBLK = _N // _RB

_ii, _jj = np.meshgrid(np.arange(_H), np.arange(_H), indexing="ij")
_iif, _jjf = _ii.ravel(), _jj.ravel()
_srcs = []
for _di, _dj in [(-1, -1), (-1, 0), (-1, 1), (0, -1), (0, 0), (0, 1),
                 (-1, -1), (-1, 0), (-1, 1)]:
    _srcs.append(((_iif + _di) % _H) * _H + (_jjf + _dj) % _H)
_src = np.concatenate(_srcs)
_dst = np.repeat((_iif * 4 + _jjf)[:, None], 9, axis=1).ravel()
_deg = np.ones(_N, dtype=np.float64)
np.add.at(_deg, _dst, 1.0)
_dinv = 1.0 / np.sqrt(_deg)

_P = np.zeros((_PAD, _N), dtype=np.float64)
np.add.at(_P, (_dst, _src), _dinv[_dst] * _dinv[_src])
_P = _P.astype(np.float32)
_DINV2 = (_dinv * _dinv).astype(np.float32).reshape(_N, 1)


def _layer_body(x_ref, p_ref, d2_ref, w_ref, b_ref, o_ref, agg_ref, *, relu):
    i = pl.program_id(0)

    @pl.when(i == 0)
    def _zero():
        agg_ref[:] = jnp.zeros_like(agg_ref)

    xb = x_ref[:]
    agg_ref[:] += jnp.dot(p_ref[:], xb, preferred_element_type=jnp.float32)
    base = jnp.dot(xb * d2_ref[:], w_ref[:],
                   preferred_element_type=jnp.float32) + b_ref[:]

    def _act(v):
        return jnp.maximum(v, 0.0) if relu else v

    @pl.when(i != _NBLK - 1)
    def _plain():
        o_ref[:] = _act(base)

    @pl.when(i == _NBLK - 1)
    def _last():
        ex = jnp.dot(agg_ref[:], w_ref[:], preferred_element_type=jnp.float32)
        full = base + jnp.concatenate(
            [ex, jnp.zeros((_RB - _PAD, ex.shape[1]), jnp.float32)], axis=0)
        o_ref[:] = _act(full)


def _layer(x, w, b, relu):
    c = x.shape[1]
    cout = w.shape[1]
    rot = lambda i: ((i + 1) % _NBLK, 0)
    return pl.pallas_call(
        functools.partial(_layer_body, relu=relu),
        grid=(_NBLK,),
        in_specs=[
            pl.BlockSpec((_RB, c), rot),
            pl.BlockSpec((_PAD, _RB), lambda i: (0, (i + 1) % _NBLK)),
            pl.BlockSpec((_RB, 1), rot),
            pl.BlockSpec((c, cout), lambda i: (0, 0)),
            pl.BlockSpec((1, cout), lambda i: (0, 0)),
        ],
        out_specs=pl.BlockSpec((_RB, cout), rot),
        out_shape=jax.ShapeDtypeStruct((_N, cout), jnp.float32),
        scratch_shapes=[pltpu.VMEM((_PAD, c), jnp.float32)],
    )(x, _P, _DINV2, w, b.reshape(1, cout))


def kernel(x, W1, b1, W2, b2, W3, b3):
    xv = x.reshape(_N, 4)
    h1 = _layer(xv, W1, b1, relu=True)
    h2 = _layer(h1, W2, b2, relu=True)
    h3 = _layer(h2, W3, b3, relu=False)
    return h3.reshape(1, _N, 512)


# speedup vs baseline: 110.9875x; 1.1878x over previous
import functools

import jax
import jax.numpy as jnp
import numpy as np
from jax.experimental import pallas as pl
from jax.experimental.pallas import tpu as pltpu

_H = 128
_N = _H * _H
_PAD = 640
_RB = 1024
_NBLK = _N // _RB

_ii, _jj = np.meshgrid(np.arange(_H), np.arange(_H), indexing="ij")
_iif, _jjf = _ii.ravel(), _jj.ravel()
_srcs = []
for _di, _dj in [(-1, -1), (-1, 0), (-1, 1), (0, -1), (0, 0), (0, 1),
                 (-1, -1), (-1, 0), (-1, 1)]:
    _srcs.append(((_iif + _di) % _H) * _H + (_jjf + _dj) % _H)
_src = np.concatenate(_srcs)
_dst = np.repeat((_iif * 4 + _jjf)[:, None], 9, axis=1).ravel()
_deg = np.ones(_N, dtype=np.float64)
np.add.at(_deg, _dst, 1.0)
_dinv = 1.0 / np.sqrt(_deg)

_P = np.zeros((_PAD, _N), dtype=np.float64)
np.add.at(_P, (_dst, _src), _dinv[_dst] * _dinv[_src])
_P16 = _P.astype(jnp.bfloat16)
_DINV2 = (_dinv * _dinv).astype(np.float32).reshape(_N, 1)


def _layer_body(x_ref, p_ref, d2_ref, w_ref, b_ref, o_ref, agg_ref, *, relu):
    i = pl.program_id(0)

    @pl.when(i == 0)
    def _zero():
        agg_ref[:] = jnp.zeros_like(agg_ref)

    xb = x_ref[:]
    agg_ref[:] += jnp.dot(p_ref[:], xb, preferred_element_type=jnp.float32)
    base = d2_ref[:] * jnp.dot(xb, w_ref[:],
                               preferred_element_type=jnp.float32)

    def _act(v):
        v = v + b_ref[:]
        return jnp.maximum(v, 0.0) if relu else v

    @pl.when(i != _NBLK - 1)
    def _plain():
        o_ref[:] = _act(base).astype(o_ref.dtype)

    @pl.when(i == _NBLK - 1)
    def _last():
        ex = jnp.dot(agg_ref[:].astype(jnp.bfloat16), w_ref[:],
                     preferred_element_type=jnp.float32)
        full = base + jnp.concatenate(
            [ex, jnp.zeros((_RB - _PAD, ex.shape[1]), jnp.float32)], axis=0)
        o_ref[:] = _act(full).astype(o_ref.dtype)


def _layer(x, w, b, relu, out_dtype):
    c = x.shape[1]
    cout = w.shape[1]
    rot = lambda i: ((i + 1) % _NBLK, 0)
    return pl.pallas_call(
        functools.partial(_layer_body, relu=relu),
        grid=(_NBLK,),
        in_specs=[
            pl.BlockSpec((_RB, c), rot),
            pl.BlockSpec((_PAD, _RB), lambda i: (0, (i + 1) % _NBLK)),
            pl.BlockSpec((_RB, 1), rot),
            pl.BlockSpec((c, cout), lambda i: (0, 0)),
            pl.BlockSpec((1, cout), lambda i: (0, 0)),
        ],
        out_specs=pl.BlockSpec((_RB, cout), rot),
        out_shape=jax.ShapeDtypeStruct((_N, cout), out_dtype),
        scratch_shapes=[pltpu.VMEM((_PAD, c), jnp.float32)],
    )(x, _P16, _DINV2, w.astype(jnp.bfloat16), b.reshape(1, cout))


def kernel(x, W1, b1, W2, b2, W3, b3):
    xv = x.reshape(_N, 4).astype(jnp.bfloat16)
    h1 = _layer(xv, W1, b1, relu=True, out_dtype=jnp.bfloat16)
    h2 = _layer(h1, W2, b2, relu=True, out_dtype=jnp.bfloat16)
    h3 = _layer(h2, W3, b3, relu=False, out_dtype=jnp.float32)
    return h3.reshape(1, _N, 512)


# speedup vs baseline: 119.2830x; 1.0747x over previous
import functools

import jax
import jax.numpy as jnp
import numpy as np
from jax.experimental import pallas as pl
from jax.experimental.pallas import tpu as pltpu

_H = 128
_N = _H * _H
_PAD = 640
_RB = 1024
_NBLK = _N // _RB

_ii, _jj = np.meshgrid(np.arange(_H), np.arange(_H), indexing="ij")
_iif, _jjf = _ii.ravel(), _jj.ravel()
_srcs = []
for _di, _dj in [(-1, -1), (-1, 0), (-1, 1), (0, -1), (0, 0), (0, 1),
                 (-1, -1), (-1, 0), (-1, 1)]:
    _srcs.append(((_iif + _di) % _H) * _H + (_jjf + _dj) % _H)
_src = np.concatenate(_srcs)
_dst = np.repeat((_iif * 4 + _jjf)[:, None], 9, axis=1).ravel()
_deg = np.ones(_N, dtype=np.float64)
np.add.at(_deg, _dst, 1.0)
_dinv = 1.0 / np.sqrt(_deg)

_P = np.zeros((_PAD, _N), dtype=np.float64)
np.add.at(_P, (_dst, _src), _dinv[_dst] * _dinv[_src])
_P16 = _P.astype(jnp.bfloat16)
_DINV2 = (_dinv * _dinv).astype(np.float32).reshape(_N, 1)


def _layer_body(x_ref, p_ref, d2_ref, w_ref, b_ref, o_ref, agg_ref, *, relu):
    i = pl.program_id(0)

    @pl.when(i == 0)
    def _zero():
        agg_ref[:] = jnp.zeros_like(agg_ref)

    xb = x_ref[:]
    agg_ref[:] += jnp.dot(p_ref[:], xb, preferred_element_type=jnp.float32)
    base = d2_ref[:] * jnp.dot(xb, w_ref[:],
                               preferred_element_type=jnp.float32)

    def _act(v):
        v = v + b_ref[:]
        return jnp.maximum(v, 0.0) if relu else v

    @pl.when(i != _NBLK - 1)
    def _plain():
        o_ref[:] = _act(base).astype(o_ref.dtype)

    @pl.when(i == _NBLK - 1)
    def _last():
        ex = jnp.dot(agg_ref[:].astype(jnp.bfloat16), w_ref[:],
                     preferred_element_type=jnp.float32)
        full = base + jnp.concatenate(
            [ex, jnp.zeros((_RB - _PAD, ex.shape[1]), jnp.float32)], axis=0)
        o_ref[:] = _act(full).astype(o_ref.dtype)


def _layer(x, w, b, relu, out_dtype):
    c = x.shape[1]
    cout = w.shape[1]
    rot = lambda i: ((i + 1) % _NBLK, 0)
    return pl.pallas_call(
        functools.partial(_layer_body, relu=relu),
        grid=(_NBLK,),
        in_specs=[
            pl.BlockSpec((_RB, c), rot),
            pl.BlockSpec((_PAD, _RB), lambda i: (0, (i + 1) % _NBLK)),
            pl.BlockSpec((_RB, 1), rot),
            pl.BlockSpec((c, cout), lambda i: (0, 0)),
            pl.BlockSpec((1, cout), lambda i: (0, 0)),
        ],
        out_specs=pl.BlockSpec((_RB, cout), rot),
        out_shape=jax.ShapeDtypeStruct((_N, cout), out_dtype),
        scratch_shapes=[pltpu.VMEM((_PAD, c), jnp.float32)],
    )(x, _P16, _DINV2, w.astype(jnp.bfloat16), b.reshape(1, cout))


def _fused23_body(h1_ref, p_ref, d2_ref, w_ref, b_ref, o_ref, h_ref, agg_ref):
    s = pl.program_id(0)
    l = s // _NBLK
    i = s % _NBLK
    r = (i + 1) % _NBLK

    @pl.when(i == 0)
    def _zero():
        agg_ref[:] = jnp.zeros_like(agg_ref)

    hs = h_ref[pl.ds(r * _RB, _RB), :]
    xb = jnp.where(l == 0, h1_ref[:], hs)
    agg_ref[:] += jnp.dot(p_ref[:, pl.ds(r * _RB, _RB)], xb,
                          preferred_element_type=jnp.float32)
    w = w_ref[0]
    base = d2_ref[:] * jnp.dot(xb, w, preferred_element_type=jnp.float32)

    def _act(v):
        return jnp.where(l == 0, jnp.maximum(v, 0.0), v)

    raw = base + b_ref[0]
    res = _act(raw)
    o_ref[:] = res

    @pl.when(l == 0)
    def _store_h():
        h_ref[pl.ds(r * _RB, _RB), :] = res.astype(jnp.bfloat16)

    @pl.when(i == _NBLK - 1)
    def _fix_top():
        ex = jnp.dot(agg_ref[:].astype(jnp.bfloat16), w,
                     preferred_element_type=jnp.float32)
        top = _act(jax.lax.slice(raw, (0, 0), (_PAD, raw.shape[1])) + ex)
        o_ref[0:_PAD, :] = top

        @pl.when(l == 0)
        def _fix_h():
            h_ref[0:_PAD, :] = top.astype(jnp.bfloat16)


def _fused23(h1, W2, b2, W3, b3):
    rotv = lambda s: ((s % _NBLK + 1) % _NBLK, 0)
    wstk = jnp.stack([W2.astype(jnp.bfloat16), W3.astype(jnp.bfloat16)])
    bstk = jnp.stack([b2, b3]).reshape(2, 1, 512)
    return pl.pallas_call(
        _fused23_body,
        grid=(2 * _NBLK,),
        in_specs=[
            pl.BlockSpec((_RB, 512),
                         lambda s: (jnp.where(s < _NBLK,
                                              (s % _NBLK + 1) % _NBLK, 0), 0)),
            pl.BlockSpec((_PAD, _N), lambda s: (0, 0)),
            pl.BlockSpec((_RB, 1), rotv),
            pl.BlockSpec((1, 512, 512), lambda s: (s // _NBLK, 0, 0)),
            pl.BlockSpec((1, 1, 512), lambda s: (s // _NBLK, 0, 0)),
        ],
        out_specs=pl.BlockSpec(
            (_RB, 512),
            lambda s: (jnp.where(s < _NBLK, 1, (s % _NBLK + 1) % _NBLK), 0)),
        out_shape=jax.ShapeDtypeStruct((_N, 512), jnp.float32),
        scratch_shapes=[pltpu.VMEM((_N, 512), jnp.bfloat16),
                        pltpu.VMEM((_PAD, 512), jnp.float32)],
    )(h1, _P16, _DINV2, wstk, bstk)


def kernel(x, W1, b1, W2, b2, W3, b3):
    xv = x.reshape(_N, 4).astype(jnp.bfloat16)
    h1 = _layer(xv, W1, b1, relu=True, out_dtype=jnp.bfloat16)
    h3 = _fused23(h1, W2, b2, W3, b3)
    return h3.reshape(1, _N, 512)


# speedup vs baseline: 134.2385x; 1.1254x over previous
import jax
import jax.numpy as jnp
import numpy as np
from jax.experimental import pallas as pl
from jax.experimental.pallas import tpu as pltpu

_H = 128
_N = _H * _H
_PAD = 640
_RB = 1024
_NBLK = _N // _RB

_ii, _jj = np.meshgrid(np.arange(_H), np.arange(_H), indexing="ij")
_iif, _jjf = _ii.ravel(), _jj.ravel()
_srcs = []
for _di, _dj in [(-1, -1), (-1, 0), (-1, 1), (0, -1), (0, 0), (0, 1),
                 (-1, -1), (-1, 0), (-1, 1)]:
    _srcs.append(((_iif + _di) % _H) * _H + (_jjf + _dj) % _H)
_src = np.concatenate(_srcs)
_dst = np.repeat((_iif * 4 + _jjf)[:, None], 9, axis=1).ravel()
_deg = np.ones(_N, dtype=np.float64)
np.add.at(_deg, _dst, 1.0)
_dinv = 1.0 / np.sqrt(_deg)

_P = np.zeros((_PAD, _N), dtype=np.float64)
np.add.at(_P, (_dst, _src), _dinv[_dst] * _dinv[_src])
_P16 = _P.astype(jnp.bfloat16)
_DINV2 = (_dinv * _dinv).astype(np.float32).reshape(_N, 1)

_F32 = jnp.float32


def _mega_body(x_ref, p_ref, d2_ref, w1_ref, wstk_ref, bstk_ref,
               o_ref, h_ref, agg_ref):
    s = pl.program_id(0)
    l = s // _NBLK
    i = s % _NBLK
    r = (i + 1) % _NBLK
    off = r * _RB

    def phase(xfull, xslice, w, relu, store):
        @pl.when(i == 0)
        def _ex():
            t = jnp.dot(p_ref[:], xfull(), preferred_element_type=_F32)
            agg_ref[:] = jnp.dot(t.astype(jnp.bfloat16), w,
                                 preferred_element_type=_F32)

        raw = (d2_ref[pl.ds(off, _RB), :] *
               jnp.dot(xslice(), w, preferred_element_type=_F32)
               ) + bstk_ref[0]
        store(jnp.maximum(raw, 0.0) if relu else raw, False)

        @pl.when(i == _NBLK - 1)
        def _top():
            top = jax.lax.slice(raw, (0, 0), (_PAD, 512)) + agg_ref[:]
            store(jnp.maximum(top, 0.0) if relu else top, True)

    def store_h(v, is_top):
        if is_top:
            h_ref[0:_PAD, :] = v.astype(jnp.bfloat16)
        else:
            h_ref[pl.ds(off, _RB), :] = v.astype(jnp.bfloat16)

    def store_o(v, is_top):
        if is_top:
            o_ref[0:_PAD, :] = v
        else:
            o_ref[:] = v

    @pl.when(l == 0)
    def _l0():
        phase(lambda: x_ref[:], lambda: x_ref[pl.ds(off, _RB), :],
              w1_ref[:], True, store_h)

    @pl.when(l == 1)
    def _l1():
        phase(lambda: h_ref[:], lambda: h_ref[pl.ds(off, _RB), :],
              wstk_ref[0], True, store_h)

    @pl.when(l == 2)
    def _l2():
        phase(lambda: h_ref[:], lambda: h_ref[pl.ds(off, _RB), :],
              wstk_ref[0], False, store_o)


def _gcn3(xv, W1, b1, W2, b2, W3, b3):
    wstk = jnp.stack([W2.astype(jnp.bfloat16), W3.astype(jnp.bfloat16)])
    bstk = jnp.stack([b1, b2, b3]).reshape(3, 1, 512)
    const = lambda s: (0, 0)
    return pl.pallas_call(
        _mega_body,
        grid=(3 * _NBLK,),
        in_specs=[
            pl.BlockSpec((_N, 4), const),
            pl.BlockSpec((_PAD, _N), const),
            pl.BlockSpec((_N, 1), const),
            pl.BlockSpec((4, 512), const),
            pl.BlockSpec((1, 512, 512),
                         lambda s: (jnp.maximum(s // _NBLK - 1, 0), 0, 0)),
            pl.BlockSpec((1, 1, 512), lambda s: (s // _NBLK, 0, 0)),
        ],
        out_specs=pl.BlockSpec(
            (_RB, 512),
            lambda s: (jnp.where(s < 2 * _NBLK, 1, (s % _NBLK + 1) % _NBLK),
                       0)),
        out_shape=jax.ShapeDtypeStruct((_N, 512), jnp.float32),
        scratch_shapes=[pltpu.VMEM((_N, 512), jnp.bfloat16),
                        pltpu.VMEM((_PAD, 512), jnp.float32)],
    )(xv, _P16, _DINV2, W1.astype(jnp.bfloat16), wstk, bstk)


def kernel(x, W1, b1, W2, b2, W3, b3):
    xv = x.reshape(_N, 4).astype(jnp.bfloat16)
    h3 = _gcn3(xv, W1, b1, W2, b2, W3, b3)
    return h3.reshape(1, _N, 512)
